# baseline (speedup 1.0000x reference)
import numpy as np
import jax
import jax.numpy as jnp
from jax.experimental import pallas as pl
from jax.experimental.pallas import tpu as pltpu

_IN = 64
_GX, _GY, _GZ = 180, 200, 5
_A = 6
_BBOX = 7
_CLS_C = _A * 4
_REG_C = _A * _BBOX
_XB = 18

_PC_RANGE = np.array([0.0, -40.0, -4.0, 72.0, 40.0, 4.0], dtype=np.float32)
_ANCHOR_SIZES = np.array([
    [3.9, 1.6, 1.56, 0.0],
    [3.9, 1.6, 1.56, 1.5707963],
    [0.8, 0.6, 1.73, 0.0],
    [0.8, 0.6, 1.73, 1.5707963],
    [1.76, 0.6, 1.73, 0.0],
    [1.76, 0.6, 1.73, 1.5707963],
], dtype=np.float32)
_STRIDE = (_PC_RANGE[3:] - _PC_RANGE[:3]) / np.array([_GX, _GY, _GZ], np.float32)

_PERM = np.array([7 * a + k for a in range(_A) for k in range(6)]
                 + [7 * a + 6 for a in range(_A)], dtype=np.int32)
_KM = np.tile(np.arange(6), _A)
_AM = np.repeat(np.arange(_A), 6)
_KM2 = np.repeat(_KM, 2)
_AM2 = np.repeat(_AM, 2)
_MULT = _ANCHOR_SIZES[_AM2, _KM2 % 3].astype(np.float32)[:, None]
_IS_XYZ = (_KM2 < 3).astype(np.float32)[:, None]
_SEL_X = (_KM2 == 0).astype(np.float32)[:, None]
_SEL_Y = (_KM2 == 1).astype(np.float32)[:, None]
_SEL_Z = (_KM2 == 2).astype(np.float32)[:, None]
_COS = np.repeat(np.cos(_ANCHOR_SIZES[:, 3]), 2).astype(np.float32)[:, None]
_TAB = np.concatenate([_MULT, _IS_XYZ, _SEL_X, _SEL_Y, _SEL_Z], axis=1)

_ASC = np.array([1.5707963050, -0.2145988016, 0.0889789874, -0.0501743046,
                 0.0308918810, -0.0170881256, 0.0066700901, -0.0012624911],
                dtype=np.float32)


def _head_kernel(x_ref, w1_ref, b1_ref, w2_ref, b2_ref, tab_ref, cos_ref,
                 cls_ref, box_ref):
    zi = pl.program_id(1)
    xrow0 = pl.program_id(0) * _XB
    cz = (_PC_RANGE[2] + _STRIDE[2] * zi.astype(jnp.float32)) + np.float32(_STRIDE[2] / 2.0)
    iy = jax.lax.broadcasted_iota(jnp.int32, (1, _GY), 1).astype(jnp.float32)
    cy = (_PC_RANGE[1] + _STRIDE[1] * iy) + np.float32(_STRIDE[1] / 2.0)

    mult = tab_ref[:, 0:1]
    is_xyz = tab_ref[:, 1:2] > 0.5
    coord_yz = cy * tab_ref[:, 3:4] + cz * tab_ref[:, 4:5]
    sel_x = tab_ref[:, 2:3]
    cosang = cos_ref[...]

    nb = x_ref.shape[0]
    nc = nb * _CLS_C
    nm = nb * 36
    for i in range(_XB):
        cx = (_PC_RANGE[0] + _STRIDE[0] * (xrow0 + i).astype(jnp.float32)) \
            + np.float32(_STRIDE[0] / 2.0)
        h = jnp.concatenate([
            jnp.maximum(
                jnp.dot(w1_ref[...], x_ref[b, i, 0].astype(jnp.bfloat16),
                        preferred_element_type=jnp.float32) + b1_ref[...], 0.0)
            for b in range(nb)], axis=0).astype(jnp.bfloat16)
        out = (jnp.dot(w2_ref[...], h, preferred_element_type=jnp.float32)
               + b2_ref[...])

        for b in range(nb):
            cls_ref[b, i, 0] = out[_CLS_C * b:_CLS_C * (b + 1)]

        om = out[nc:nc + nm]
        dec_main = jnp.where(is_xyz, om * mult + (coord_yz + cx * sel_x),
                             jnp.exp(om) * mult)
        c = jax.nn.sigmoid(out[nc + nm:]) * cosang
        p = _ASC[7]
        for j in range(6, -1, -1):
            p = p * c + _ASC[j]
        ang = np.float32(np.pi / 2) - jnp.sqrt(1.0 - c) * p

        pieces = []
        for a in range(_A):
            pieces.append(dec_main[12 * a:12 * a + 12])
            pieces.append(ang[2 * a:2 * a + 2])
        box_ref[i, 0] = jnp.concatenate(pieces, axis=0).reshape(_REG_C, nb, _GY)


def kernel(x, W1c, b1c, gc, bec, mc, vc, W2c, b2c, W1r, b1r, gr, ber, mr, vr, W2r, b2r):
    B = x.shape[0]
    xt = jnp.transpose(x, (0, 2, 4, 1, 3))

    sc = gc * jax.lax.rsqrt(vc + 1e-5)
    sr = gr * jax.lax.rsqrt(vr + 1e-5)
    w1 = jnp.concatenate([W1c * sc[:, None], W1r * sr[:, None]], axis=0)
    b1 = jnp.concatenate([b1c * sc + (bec - mc * sc),
                          b1r * sr + (ber - mr * sr)])[:, None]

    perm = jnp.asarray(_PERM)
    w2r_p = W2r[perm]
    b2r_p = b2r[perm]
    nrows = B * (_CLS_C + _REG_C)
    w2 = jnp.zeros((nrows, 2 * _IN * B), jnp.float32)
    b2 = jnp.zeros((nrows,), jnp.float32)
    for b in range(B):
        w2 = w2.at[_CLS_C * b:_CLS_C * (b + 1),
                   2 * _IN * b:2 * _IN * b + _IN].set(W2c)
        b2 = b2.at[_CLS_C * b:_CLS_C * (b + 1)].set(b2c)
        w2 = w2.at[B * _CLS_C + b:B * _CLS_C + B * 36:B,
                   2 * _IN * b + _IN:2 * _IN * (b + 1)].set(w2r_p[0:36])
        b2 = b2.at[B * _CLS_C + b:B * _CLS_C + B * 36:B].set(b2r_p[0:36])
        w2 = w2.at[B * (_CLS_C + 36) + b::B,
                   2 * _IN * b + _IN:2 * _IN * (b + 1)].set(w2r_p[36:42])
        b2 = b2.at[B * (_CLS_C + 36) + b::B].set(b2r_p[36:42])
    b2 = b2[:, None]

    grid = (_GX // _XB, _GZ)
    full = lambda shape: pl.BlockSpec(shape, lambda xi, zi: (0,) * len(shape))
    cls_t, box_t = pl.pallas_call(
        _head_kernel,
        grid=grid,
        in_specs=[
            pl.BlockSpec((B, _XB, 1, _IN, _GY), lambda xi, zi: (0, xi, zi, 0, 0)),
            full((2 * _IN, _IN)), full((2 * _IN, 1)),
            full((nrows, 2 * _IN * B)), full((nrows, 1)),
            full((B * 36, 5)), full((B * _A, 1)),
        ],
        out_specs=[
            pl.BlockSpec((B, _XB, 1, _CLS_C, _GY), lambda xi, zi: (0, xi, zi, 0, 0)),
            pl.BlockSpec((_XB, 1, _REG_C, B, _GY), lambda xi, zi: (xi, zi, 0, 0, 0)),
        ],
        out_shape=[
            jax.ShapeDtypeStruct((B, _GX, _GZ, _CLS_C, _GY), jnp.float32),
            jax.ShapeDtypeStruct((_GX, _GZ, _REG_C, B, _GY), jnp.float32),
        ],
        compiler_params=pltpu.CompilerParams(
            dimension_semantics=("parallel", "parallel")),
    )(xt, w1.astype(jnp.bfloat16), b1, w2.astype(jnp.bfloat16), b2,
      jnp.asarray(_TAB), jnp.asarray(_COS))

    cls_out = jnp.transpose(cls_t, (0, 1, 4, 2, 3))
    box_out = jnp.transpose(box_t, (3, 0, 4, 1, 2))
    return cls_out, box_out


# speedup vs baseline: 1.7645x; 1.7645x over previous
import numpy as np
import jax
import jax.numpy as jnp
from jax.experimental import pallas as pl
from jax.experimental.pallas import tpu as pltpu

_IN = 64
_GX, _GY, _GZ = 180, 200, 5
_A = 6
_BBOX = 7
_CLS_C = _A * 4
_REG_C = _A * _BBOX
_XB = 18

_PC_RANGE = np.array([0.0, -40.0, -4.0, 72.0, 40.0, 4.0], dtype=np.float32)
_ANCHOR_SIZES = np.array([
    [3.9, 1.6, 1.56, 0.0],
    [3.9, 1.6, 1.56, 1.5707963],
    [0.8, 0.6, 1.73, 0.0],
    [0.8, 0.6, 1.73, 1.5707963],
    [1.76, 0.6, 1.73, 0.0],
    [1.76, 0.6, 1.73, 1.5707963],
], dtype=np.float32)
_STRIDE = (_PC_RANGE[3:] - _PC_RANGE[:3]) / np.array([_GX, _GY, _GZ], np.float32)

_PERM = np.array([7 * a + k for a in range(_A) for k in range(6)]
                 + [7 * a + 6 for a in range(_A)], dtype=np.int32)
_KM = np.tile(np.arange(6), _A)
_AM = np.repeat(np.arange(_A), 6)
_MULT = _ANCHOR_SIZES[_AM, _KM % 3].astype(np.float32)[:, None]
_IS_XYZ = (_KM < 3).astype(np.float32)[:, None]
_SEL_X = (_KM == 0).astype(np.float32)[:, None]
_SEL_Y = (_KM == 1).astype(np.float32)[:, None]
_SEL_Z = (_KM == 2).astype(np.float32)[:, None]
_COS = np.cos(_ANCHOR_SIZES[:, 3]).astype(np.float32)[:, None]
_TAB = np.concatenate([_MULT, _IS_XYZ, _SEL_X, _SEL_Y, _SEL_Z], axis=1)


def _head_kernel(x_ref, w1c_ref, b1c_ref, w2c_ref, b2c_ref,
                 w1r_ref, b1r_ref, w2r_ref, b2r_ref, tab_ref, cos_ref,
                 cls_ref, box_ref):
    zi = pl.program_id(1)
    xrow0 = pl.program_id(0) * _XB
    cz = (_PC_RANGE[2] + _STRIDE[2] * zi.astype(jnp.float32)) + np.float32(_STRIDE[2] / 2.0)
    iy = jax.lax.broadcasted_iota(jnp.int32, (1, _GY), 1).astype(jnp.float32)
    cy = (_PC_RANGE[1] + _STRIDE[1] * iy) + np.float32(_STRIDE[1] / 2.0)

    mult = tab_ref[:, 0:1]
    is_xyz = tab_ref[:, 1:2] > 0.5
    coord_yz = cy * tab_ref[:, 3:4] + cz * tab_ref[:, 4:5]
    sel_x = tab_ref[:, 2:3]
    cosang = cos_ref[...]

    nb = x_ref.shape[0]
    for i in range(_XB):
        cx = (_PC_RANGE[0] + _STRIDE[0] * (xrow0 + i).astype(jnp.float32)) \
            + np.float32(_STRIDE[0] / 2.0)
        dec_mains, angs = [], []
        for b in range(nb):
            xb = x_ref[b, i, 0]
            hc = jnp.maximum(
                jnp.dot(w1c_ref[...], xb, preferred_element_type=jnp.float32)
                + b1c_ref[...], 0.0)
            cls_ref[b, i, 0] = (
                jnp.dot(w2c_ref[...], hc, preferred_element_type=jnp.float32)
                + b2c_ref[...])

            hr = jnp.maximum(
                jnp.dot(w1r_ref[...], xb, preferred_element_type=jnp.float32)
                + b1r_ref[...], 0.0)
            off = (jnp.dot(w2r_ref[...], hr, preferred_element_type=jnp.float32)
                   + b2r_ref[...])

            om = off[0:36]
            dec_mains.append(jnp.where(
                is_xyz, om * mult + (coord_yz + cx * sel_x),
                jnp.exp(om) * mult)[:, None, :])
            c = jax.nn.sigmoid(off[36:42]) * cosang
            angs.append(jnp.arctan2(c, jnp.sqrt(1.0 - c * c))[:, None, :])

        sm = jnp.concatenate(dec_mains, axis=1)
        sa = jnp.concatenate(angs, axis=1)
        pieces = []
        for a in range(_A):
            pieces.append(sm[6 * a:6 * a + 6])
            pieces.append(sa[a:a + 1])
        box_ref[i, 0] = jnp.concatenate(pieces, axis=0)


def kernel(x, W1c, b1c, gc, bec, mc, vc, W2c, b2c, W1r, b1r, gr, ber, mr, vr, W2r, b2r):
    B = x.shape[0]
    xt = jnp.transpose(x, (0, 2, 4, 1, 3))

    sc = gc * jax.lax.rsqrt(vc + 1e-5)
    w1c_eff = W1c * sc[:, None]
    b1c_eff = (b1c * sc + (bec - mc * sc))[:, None]
    sr = gr * jax.lax.rsqrt(vr + 1e-5)
    w1r_eff = W1r * sr[:, None]
    b1r_eff = (b1r * sr + (ber - mr * sr))[:, None]
    perm = jnp.asarray(_PERM)

    grid = (_GX // _XB, _GZ)
    full = lambda shape: pl.BlockSpec(shape, lambda xi, zi: (0,) * len(shape))
    cls_t, box_t = pl.pallas_call(
        _head_kernel,
        grid=grid,
        in_specs=[
            pl.BlockSpec((B, _XB, 1, _IN, _GY), lambda xi, zi: (0, xi, zi, 0, 0)),
            full((_IN, _IN)), full((_IN, 1)), full((_CLS_C, _IN)), full((_CLS_C, 1)),
            full((_IN, _IN)), full((_IN, 1)), full((_REG_C, _IN)), full((_REG_C, 1)),
            full((36, 5)), full((_A, 1)),
        ],
        out_specs=[
            pl.BlockSpec((B, _XB, 1, _CLS_C, _GY), lambda xi, zi: (0, xi, zi, 0, 0)),
            pl.BlockSpec((_XB, 1, _REG_C, B, _GY), lambda xi, zi: (xi, zi, 0, 0, 0)),
        ],
        out_shape=[
            jax.ShapeDtypeStruct((B, _GX, _GZ, _CLS_C, _GY), jnp.float32),
            jax.ShapeDtypeStruct((_GX, _GZ, _REG_C, B, _GY), jnp.float32),
        ],
        compiler_params=pltpu.CompilerParams(
            dimension_semantics=("parallel", "parallel")),
    )(xt, w1c_eff, b1c_eff, W2c, b2c[:, None],
      w1r_eff, b1r_eff, W2r[perm], b2r[perm][:, None],
      jnp.asarray(_TAB), jnp.asarray(_COS))

    cls_out = jnp.transpose(cls_t, (0, 1, 4, 2, 3))
    box_out = jnp.transpose(box_t, (3, 0, 4, 1, 2))
    return cls_out, box_out
